# baseline (speedup 1.0000x reference)
import functools

import jax
import jax.numpy as jnp
from jax import lax
from jax.experimental import pallas as pl
from jax.experimental.pallas import tpu as pltpu
from jax.experimental.pallas import tpu_sc as plsc

NC = 2
NS = 16
NW = NC * NS

B = 1024
L = 200
D = 64
V = 1_000_000

WB = 256
NB = V // WB
VMAIN = NB * WB
PKROWS = VMAIN // 2
NB_W = NB // NW
NB_X = NB - NW * NB_W
CHUNK = 128
NCH = B // CHUNK

_params = pltpu.CompilerParams(use_tc_tiling_on_sc=True, needs_layout_passes=False)


def _mesh():
    return plsc.VectorSubcoreMesh(
        core_axis_name="c", subcore_axis_name="s", num_cores=NC, num_subcores=NS
    )


def _wid():
    return lax.axis_index("s") * NC + lax.axis_index("c")


def _transpose_block(in_v, out_v):
    iota = lax.iota(jnp.int32, 16)
    iota_half = lax.shift_right_logical(iota, 1)
    col_base = lax.shift_left(jnp.bitwise_and(iota, 1), 6)
    rows = [iota_half + 8 * g for g in range(WB // 16)]

    def d_body(d, _):
        col = col_base + d
        for g in range(WB // 16):
            val = in_v[d, pl.ds(16 * g, 16)]
            plsc.store_scatter(out_v, [rows[g], col], val)
        return ()

    lax.fori_loop(0, D, d_body, (), unroll=2)


def _pack_body(tok_t, pk, in_a, in_b, out_a, out_b, si_a, si_b, so_a, so_b):
    wid = _wid()
    base = wid * NB_W

    def start_in(i, buf, sem):
        pltpu.async_copy(tok_t.at[:, pl.ds(i * WB, WB)], buf, sem)

    def start_out(i, buf, sem):
        pltpu.async_copy(buf, pk.at[pl.ds(i * (WB // 2), WB // 2), :], sem)

    def wait_in(buf, sem):
        pltpu.make_async_copy(tok_t.at[:, pl.ds(0, WB)], buf, sem).wait()

    def wait_out(buf, sem):
        pltpu.make_async_copy(buf, pk.at[pl.ds(0, WB // 2), :], sem).wait()

    start_in(base, in_a, si_a)

    def u_body(u, _):
        t_a = base + 2 * u
        start_in(t_a + 1, in_b, si_b)
        with jax.named_scope("pk_wait_in_a"):
            wait_in(in_a, si_a)

        @pl.when(u > 0)
        def _():
            with jax.named_scope("pk_wait_out_a"):
                wait_out(out_a, so_a)

        with jax.named_scope("pk_transpose_a"):
            _transpose_block(in_a, out_a)
        start_out(t_a, out_a, so_a)

        @pl.when(u < NB_W // 2 - 1)
        def _():
            start_in(t_a + 2, in_a, si_a)

        wait_in(in_b, si_b)

        @pl.when(u > 0)
        def _():
            wait_out(out_b, so_b)

        _transpose_block(in_b, out_b)
        start_out(t_a + 1, out_b, so_b)
        return ()

    lax.fori_loop(0, NB_W // 2, u_body, ())

    @pl.when(wid < NB_X)
    def _extra():
        i = NW * NB_W + wid
        wait_out(out_a, so_a)
        pltpu.sync_copy(tok_t.at[:, pl.ds(i * WB, WB)], in_a)
        _transpose_block(in_a, out_a)
        start_out(i, out_a, so_a)

    @pl.when(wid >= NB_X)
    def _():
        wait_out(out_a, so_a)

    @pl.when(wid < NB_X)
    def _():
        wait_out(out_a, so_a)

    wait_out(out_b, so_b)


def _gather_body(pk, x_t, pos_hbm, aux_hbm, out3,
                 pos_v, g_a, g_b, ostage, xi_l, idx2_l, offs_l, rowsel_l,
                 sg_a, sg_b):
    wid = _wid()
    iota = lax.iota(jnp.int32, 16)

    pltpu.sync_copy(pos_hbm, pos_v)
    pltpu.sync_copy(aux_hbm, g_a.at[pl.ds(CHUNK, 32), :])
    pltpu.sync_copy(aux_hbm, g_b.at[pl.ds(CHUNK, 32), :])
    nl = jnp.where(wid < L - NW * (L // NW), L // NW + 1, L // NW)

    def start_g(c, buf, sem):
        pltpu.async_copy(
            pk.at[idx2_l.at[pl.ds(CHUNK * c, CHUNK)]],
            buf.at[pl.ds(0, CHUNK), :], sem)

    def wait_g(buf, sem):
        pltpu.make_async_copy(
            pk.at[idx2_l.at[pl.ds(0, CHUNK)]],
            buf.at[pl.ds(0, CHUNK), :], sem).wait()

    def l_body(j, _):
        l = wid + NW * j
        with jax.named_scope("g_xi"):
            pltpu.sync_copy(x_t.at[l], xi_l)

        def prep(k, _):
            sl = pl.ds(16 * k, 16)
            xi = xi_l[sl]
            idx2 = lax.shift_right_logical(xi, 1)
            tail = idx2 >= PKROWS
            idx2_l[sl] = jnp.where(tail, PKROWS - 1, idx2)
            offs_l[sl] = lax.shift_left(jnp.bitwise_and(xi, 1), 6)
            rowsel_l[sl] = jnp.where(
                tail, idx2 - (PKROWS - CHUNK), ((16 * k) % CHUNK) + iota)
            return ()

        with jax.named_scope("g_prep"):
            lax.fori_loop(0, B // 16, prep, ())
        start_g(0, g_a, sg_a)

        def compute(buf, c):
            cb = CHUNK * c
            rows = [rowsel_l[pl.ds(cb + 16 * g, 16)] for g in range(CHUNK // 16)]
            offs = [offs_l[pl.ds(cb + 16 * g, 16)] for g in range(CHUNK // 16)]
            pbase = jnp.full((16,), l * D, jnp.int32)

            def d_body(d, _):
                pv = plsc.load_gather(pos_v, [pbase + d])
                for g in range(CHUNK // 16):
                    val = plsc.load_gather(buf, [rows[g], offs[g] + d])
                    ostage[d, pl.ds(cb + 16 * g, 16)] = val + pv
                return ()

            lax.fori_loop(0, D, d_body, (), unroll=2)

        def c_body(p, _):
            start_g(2 * p + 1, g_b, sg_b)
            with jax.named_scope("g_wait_a"):
                wait_g(g_a, sg_a)
            with jax.named_scope("g_compute_a"):
                compute(g_a, 2 * p)

            @pl.when(p < NCH // 2 - 1)
            def _():
                start_g(2 * p + 2, g_a, sg_a)

            wait_g(g_b, sg_b)
            compute(g_b, 2 * p + 1)
            return ()

        lax.fori_loop(0, NCH // 2, c_body, ())
        with jax.named_scope("g_out"):
            pltpu.sync_copy(ostage, out3.at[l])
        return ()

    lax.fori_loop(0, nl, l_body, ())


@jax.jit
def _run(tok_t, x_t, pos_flat, aux_pairs):
    pack = pl.kernel(
        _pack_body,
        out_type=jax.ShapeDtypeStruct((PKROWS, 128), jnp.float32),
        mesh=_mesh(),
        scratch_types=[
            pltpu.VMEM((D, WB), jnp.float32),
            pltpu.VMEM((D, WB), jnp.float32),
            pltpu.VMEM((WB // 2, 128), jnp.float32),
            pltpu.VMEM((WB // 2, 128), jnp.float32),
            pltpu.SemaphoreType.DMA,
            pltpu.SemaphoreType.DMA,
            pltpu.SemaphoreType.DMA,
            pltpu.SemaphoreType.DMA,
        ],
        compiler_params=_params,
    )
    pk = pack(tok_t)

    gather = pl.kernel(
        _gather_body,
        out_type=jax.ShapeDtypeStruct((L, D, B), jnp.float32),
        mesh=_mesh(),
        scratch_types=[
            pltpu.VMEM((L * D,), jnp.float32),
            pltpu.VMEM((CHUNK + 32, 128), jnp.float32),
            pltpu.VMEM((CHUNK + 32, 128), jnp.float32),
            pltpu.VMEM((D, B), jnp.float32),
            pltpu.VMEM((B,), jnp.int32),
            pltpu.VMEM((B,), jnp.int32),
            pltpu.VMEM((B,), jnp.int32),
            pltpu.VMEM((B,), jnp.int32),
            pltpu.SemaphoreType.DMA,
            pltpu.SemaphoreType.DMA,
        ],
        compiler_params=_params,
    )
    out3 = gather(pk, x_t, pos_flat, aux_pairs)
    return out3.transpose(2, 0, 1)


def kernel(x, token_table, pos_table):
    tok_t = token_table.T
    x_t = x.T.astype(jnp.int32)
    pos_flat = pos_table.reshape(-1)
    aux_pairs = lax.slice(token_table, (VMAIN, 0), (V, D)).reshape(32, 128)
    return _run(tok_t, x_t, pos_flat, aux_pairs)


# speedup vs baseline: 1.0491x; 1.0491x over previous
import functools

import jax
import jax.numpy as jnp
from jax import lax
from jax.experimental import pallas as pl
from jax.experimental.pallas import tpu as pltpu
from jax.experimental.pallas import tpu_sc as plsc

NC = 2
NS = 16
NW = NC * NS

B = 1024
L = 200
D = 64
V = 1_000_000

WB = 256
NB = V // WB
VMAIN = NB * WB
PKROWS = VMAIN // 2
NB_W = NB // NW
NB_X = NB - NW * NB_W
CHUNK = 128
NCH = B // CHUNK

_params = pltpu.CompilerParams(use_tc_tiling_on_sc=True, needs_layout_passes=False)


def _mesh():
    return plsc.VectorSubcoreMesh(
        core_axis_name="c", subcore_axis_name="s", num_cores=NC, num_subcores=NS
    )


def _wid():
    return lax.axis_index("s") * NC + lax.axis_index("c")


def _transpose_block(in_v, out_v):
    iota = lax.iota(jnp.int32, 16)
    iota_half = lax.shift_right_logical(iota, 1)
    col_base = lax.shift_left(jnp.bitwise_and(iota, 1), 6)
    rows = [iota_half + 8 * g for g in range(WB // 16)]

    def d_body(d, _):
        col = col_base + d
        vals = [in_v[d, pl.ds(16 * g, 16)] for g in range(WB // 16)]
        for g in range(WB // 16):
            plsc.store_scatter(out_v, [rows[g], col], vals[g])
        return ()

    lax.fori_loop(0, D, d_body, (), unroll=2)


def _pack_body(tok_t, pk, in_a, in_b, out_a, out_b, si_a, si_b, so_a, so_b):
    wid = _wid()
    base = wid * NB_W

    def start_in(i, buf, sem):
        pltpu.async_copy(tok_t.at[:, pl.ds(i * WB, WB)], buf, sem)

    def start_out(i, buf, sem):
        pltpu.async_copy(buf, pk.at[pl.ds(i * (WB // 2), WB // 2), :], sem)

    def wait_in(buf, sem):
        pltpu.make_async_copy(tok_t.at[:, pl.ds(0, WB)], buf, sem).wait()

    def wait_out(buf, sem):
        pltpu.make_async_copy(buf, pk.at[pl.ds(0, WB // 2), :], sem).wait()

    start_in(base, in_a, si_a)

    def u_body(u, _):
        t_a = base + 2 * u
        start_in(t_a + 1, in_b, si_b)
        with jax.named_scope("pk_wait_in_a"):
            wait_in(in_a, si_a)

        @pl.when(u > 0)
        def _():
            with jax.named_scope("pk_wait_out_a"):
                wait_out(out_a, so_a)

        with jax.named_scope("pk_transpose_a"):
            _transpose_block(in_a, out_a)
        start_out(t_a, out_a, so_a)

        @pl.when(u < NB_W // 2 - 1)
        def _():
            start_in(t_a + 2, in_a, si_a)

        wait_in(in_b, si_b)

        @pl.when(u > 0)
        def _():
            wait_out(out_b, so_b)

        _transpose_block(in_b, out_b)
        start_out(t_a + 1, out_b, so_b)
        return ()

    lax.fori_loop(0, NB_W // 2, u_body, ())

    @pl.when(wid < NB_X)
    def _extra():
        i = NW * NB_W + wid
        wait_out(out_a, so_a)
        pltpu.sync_copy(tok_t.at[:, pl.ds(i * WB, WB)], in_a)
        _transpose_block(in_a, out_a)
        start_out(i, out_a, so_a)

    @pl.when(wid >= NB_X)
    def _():
        wait_out(out_a, so_a)

    @pl.when(wid < NB_X)
    def _():
        wait_out(out_a, so_a)

    wait_out(out_b, so_b)


def _gather_body(pk, x_t, pos_hbm, aux_hbm, out3,
                 pos_v, g_a, g_b, ostage, xi_l, idx2_l, offs_l, rowsel_l,
                 sg_a, sg_b):
    wid = _wid()
    iota = lax.iota(jnp.int32, 16)

    pltpu.sync_copy(pos_hbm, pos_v)
    pltpu.sync_copy(aux_hbm, g_a.at[pl.ds(CHUNK, 32), :])
    pltpu.sync_copy(aux_hbm, g_b.at[pl.ds(CHUNK, 32), :])
    nl = jnp.where(wid < L - NW * (L // NW), L // NW + 1, L // NW)

    def start_g(c, buf, sem):
        pltpu.async_copy(
            pk.at[idx2_l.at[pl.ds(CHUNK * c, CHUNK)]],
            buf.at[pl.ds(0, CHUNK), :], sem)

    def wait_g(buf, sem):
        pltpu.make_async_copy(
            pk.at[idx2_l.at[pl.ds(0, CHUNK)]],
            buf.at[pl.ds(0, CHUNK), :], sem).wait()

    def l_body(j, _):
        l = wid + NW * j
        with jax.named_scope("g_xi"):
            pltpu.sync_copy(x_t.at[l], xi_l)

        def prep(k, _):
            sl = pl.ds(16 * k, 16)
            xi = xi_l[sl]
            idx2 = lax.shift_right_logical(xi, 1)
            tail = idx2 >= PKROWS
            idx2_l[sl] = jnp.where(tail, PKROWS - 1, idx2)
            offs_l[sl] = lax.shift_left(jnp.bitwise_and(xi, 1), 6)
            rowsel_l[sl] = jnp.where(
                tail, idx2 - (PKROWS - CHUNK), ((16 * k) % CHUNK) + iota)
            return ()

        with jax.named_scope("g_prep"):
            lax.fori_loop(0, B // 16, prep, ())
        start_g(0, g_a, sg_a)

        def compute(buf, c):
            cb = CHUNK * c
            rows = [rowsel_l[pl.ds(cb + 16 * g, 16)] for g in range(CHUNK // 16)]
            offs = [offs_l[pl.ds(cb + 16 * g, 16)] for g in range(CHUNK // 16)]
            pbase = jnp.full((16,), l * D, jnp.int32)

            def d_body(d, _):
                pv = plsc.load_gather(pos_v, [pbase + d])
                vals = [plsc.load_gather(buf, [rows[g], offs[g] + d])
                        for g in range(CHUNK // 16)]
                for g in range(CHUNK // 16):
                    ostage[d, pl.ds(cb + 16 * g, 16)] = vals[g] + pv
                return ()

            lax.fori_loop(0, D, d_body, (), unroll=2)

        def c_body(p, _):
            start_g(2 * p + 1, g_b, sg_b)
            with jax.named_scope("g_wait_a"):
                wait_g(g_a, sg_a)
            with jax.named_scope("g_compute_a"):
                compute(g_a, 2 * p)

            @pl.when(p < NCH // 2 - 1)
            def _():
                start_g(2 * p + 2, g_a, sg_a)

            wait_g(g_b, sg_b)
            compute(g_b, 2 * p + 1)
            return ()

        lax.fori_loop(0, NCH // 2, c_body, ())
        with jax.named_scope("g_out"):
            pltpu.sync_copy(ostage, out3.at[l])
        return ()

    lax.fori_loop(0, nl, l_body, ())


@jax.jit
def _run(tok_t, x_t, pos_flat, aux_pairs):
    pack = pl.kernel(
        _pack_body,
        out_type=jax.ShapeDtypeStruct((PKROWS, 128), jnp.float32),
        mesh=_mesh(),
        scratch_types=[
            pltpu.VMEM((D, WB), jnp.float32),
            pltpu.VMEM((D, WB), jnp.float32),
            pltpu.VMEM((WB // 2, 128), jnp.float32),
            pltpu.VMEM((WB // 2, 128), jnp.float32),
            pltpu.SemaphoreType.DMA,
            pltpu.SemaphoreType.DMA,
            pltpu.SemaphoreType.DMA,
            pltpu.SemaphoreType.DMA,
        ],
        compiler_params=_params,
    )
    pk = pack(tok_t)

    gather = pl.kernel(
        _gather_body,
        out_type=jax.ShapeDtypeStruct((L, D, B), jnp.float32),
        mesh=_mesh(),
        scratch_types=[
            pltpu.VMEM((L * D,), jnp.float32),
            pltpu.VMEM((CHUNK + 32, 128), jnp.float32),
            pltpu.VMEM((CHUNK + 32, 128), jnp.float32),
            pltpu.VMEM((D, B), jnp.float32),
            pltpu.VMEM((B,), jnp.int32),
            pltpu.VMEM((B,), jnp.int32),
            pltpu.VMEM((B,), jnp.int32),
            pltpu.VMEM((B,), jnp.int32),
            pltpu.SemaphoreType.DMA,
            pltpu.SemaphoreType.DMA,
        ],
        compiler_params=_params,
    )
    out3 = gather(pk, x_t, pos_flat, aux_pairs)
    return out3.transpose(2, 0, 1)


def kernel(x, token_table, pos_table):
    tok_t = token_table.T
    x_t = x.T.astype(jnp.int32)
    pos_flat = pos_table.reshape(-1)
    aux_pairs = lax.slice(token_table, (VMAIN, 0), (V, D)).reshape(32, 128)
    return _run(tok_t, x_t, pos_flat, aux_pairs)
